# baseline (speedup 1.0000x reference)
import jax
import jax.numpy as jnp
from jax import lax
from jax.experimental import pallas as pl
from jax.experimental.pallas import tpu as pltpu
from jax.experimental.pallas import tpu_sc as plsc

N_AGENTS = 100000
N_EDGES = 3200000
N_STEPS = 10

NC = 2
NT = 16
NW = NC * NT
NPT = 6272
NPAD = NT * NPT
NPT2 = NPAD // NW
NVEC2 = NPT2 // 16
EPW = 100352
EPAD = NW * EPW
CH = 3584
NCHUNK = EPW // CH
NOUT = NCHUNK // 4
NSUM = 3 * 16

_mesh = plsc.VectorSubcoreMesh(core_axis_name="c", subcore_axis_name="s")
_f32 = jnp.float32


def _sd(shape):
    return jax.ShapeDtypeStruct(shape, _f32)


def _edge_body(with_deg, src_h, dst_h, inf_h, pacc_h, *rest):
    if with_deg:
        pdeg_h, *rest = rest
    (sidx0, didx0, sidx1, didx1, sidx2, didx2, sidx3, didx3,
     vals0, vals1, ones_v, zero_l, acc_s, deg_s,
     sem0, sem1, sem2, sem3, ssc0, ssc1) = rest

    cid = lax.axis_index("c")
    sid = lax.axis_index("s")
    wid = cid * NT + sid
    nbase = sid * NPT
    ebase = wid * EPW

    def fill(ref, n, val):
        def b(i, car):
            ref[pl.ds(i * 16, 16)] = jnp.full((16,), val, _f32)
            return car
        lax.fori_loop(0, n // 16, b, 0)

    if with_deg:
        fill(ones_v, CH, 1.0)
    fill(zero_l, NPT, 0.0)

    pltpu.sync_copy(zero_l, acc_s.at[pl.ds(nbase, NPT)])
    if with_deg:
        pltpu.sync_copy(zero_l, deg_s.at[pl.ds(nbase, NPT)])
    plsc.subcore_barrier()

    ibufs = [(sidx0, didx0, sem0), (sidx1, didx1, sem1),
             (sidx2, didx2, sem2), (sidx3, didx3, sem3)]
    vbufs = [(vals0, ssc0), (vals1, ssc1)]

    def start_idx(c, b):
        sb, db, sem = ibufs[b]
        base = ebase + c * CH
        pltpu.make_async_copy(src_h.at[pl.ds(base, CH)], sb, sem).start()
        pltpu.make_async_copy(dst_h.at[pl.ds(base, CH)], db, sem).start()

    def wait_idx(c, b):
        sb, db, sem = ibufs[b]
        base = ebase + c * CH
        pltpu.make_async_copy(src_h.at[pl.ds(base, CH)], sb, sem).wait()
        pltpu.make_async_copy(dst_h.at[pl.ds(base, CH)], db, sem).wait()

    def wait_sc(v):
        vb, sem = vbufs[v]
        pltpu.make_async_copy(vb, acc_s.at[ibufs[0][1]], sem).wait()

    start_idx(0, 0)
    start_idx(1, 1)

    def edge_loop(co, car):
        for u in range(4):
            e = 4 * co + u
            vb, vsem = vbufs[u % 2]

            @pl.when(e >= 2)
            def _():
                wait_sc(u % 2)

            wait_idx(e, u)
            pltpu.sync_copy(inf_h.at[ibufs[u][0]], vb)
            pltpu.make_async_copy(
                vb, acc_s.at[ibufs[u][1]], vsem).start(add=True)
            if with_deg:
                pltpu.sync_copy(ones_v, deg_s.at[ibufs[u][1]], add=True)

            @pl.when(e + 2 < NCHUNK)
            def _():
                start_idx(e + 2, (u + 2) % 4)
        return car

    lax.fori_loop(0, NOUT, edge_loop, 0)
    wait_sc(0)
    wait_sc(1)
    plsc.subcore_barrier()

    pltpu.sync_copy(acc_s.at[pl.ds(nbase, NPT)],
                    pacc_h.at[pl.ds(cid * NPAD + nbase, NPT)])
    if with_deg:
        pltpu.sync_copy(deg_s.at[pl.ds(nbase, NPT)],
                        pdeg_h.at[pl.ds(cid * NPAD + nbase, NPT)])


_edge_scratch = [
    pltpu.VMEM((CH,), jnp.int32),
    pltpu.VMEM((CH,), jnp.int32),
    pltpu.VMEM((CH,), jnp.int32),
    pltpu.VMEM((CH,), jnp.int32),
    pltpu.VMEM((CH,), jnp.int32),
    pltpu.VMEM((CH,), jnp.int32),
    pltpu.VMEM((CH,), jnp.int32),
    pltpu.VMEM((CH,), jnp.int32),
    pltpu.VMEM((CH,), _f32),
    pltpu.VMEM((CH,), _f32),
    pltpu.VMEM((CH,), _f32),
    pltpu.VMEM((NPT,), _f32),
    pltpu.VMEM_SHARED((NPAD,), _f32),
    pltpu.VMEM_SHARED((NPAD,), _f32),
    pltpu.SemaphoreType.DMA,
    pltpu.SemaphoreType.DMA,
    pltpu.SemaphoreType.DMA,
    pltpu.SemaphoreType.DMA,
    pltpu.SemaphoreType.DMA,
    pltpu.SemaphoreType.DMA,
]

_edge0 = pl.kernel(
    lambda *a: _edge_body(True, *a), mesh=_mesh,
    out_type=(_sd((NC * NPAD,)), _sd((NC * NPAD,))),
    scratch_types=_edge_scratch)

_edge = pl.kernel(
    lambda *a: _edge_body(False, *a), mesh=_mesh,
    out_type=_sd((NC * NPAD,)),
    scratch_types=_edge_scratch)


def _init_body(thr_h, frac_h, inf_h, sus_h, rec_h, sums_h,
               thra, inf_l, sus_l, rec_l, sums_v, frac_v):
    cid = lax.axis_index("c")
    sid = lax.axis_index("s")
    wid = cid * NT + sid
    nb = wid * NPT2

    pltpu.sync_copy(thr_h.at[pl.ds(nb, NPT2)], thra)
    pltpu.sync_copy(frac_h, frac_v)
    fv = frac_v[...]

    def b(i, car):
        ssus, sinf = car
        t0 = thra[pl.ds(i * 16, 16)]
        inf = jnp.where(fv >= t0, 1.0, 0.0)
        sus = jnp.where(t0 > 1.5, 0.0, 1.0 - inf)
        inf_l[pl.ds(i * 16, 16)] = inf
        sus_l[pl.ds(i * 16, 16)] = sus
        rec_l[pl.ds(i * 16, 16)] = jnp.zeros((16,), _f32)
        return (ssus + sus, sinf + inf)

    z16 = jnp.zeros((16,), _f32)
    ssus, sinf = lax.fori_loop(0, NVEC2, b, (z16, z16))
    sums_v[pl.ds(0, 16)] = ssus
    sums_v[pl.ds(16, 16)] = sinf
    sums_v[pl.ds(32, 16)] = z16
    pltpu.sync_copy(inf_l, inf_h.at[pl.ds(nb, NPT2)])
    pltpu.sync_copy(sus_l, sus_h.at[pl.ds(nb, NPT2)])
    pltpu.sync_copy(rec_l, rec_h.at[pl.ds(nb, NPT2)])
    pltpu.sync_copy(sums_v, sums_h.at[pl.ds(wid * NSUM, NSUM)])


_init = pl.kernel(
    _init_body, mesh=_mesh,
    out_type=(_sd((NPAD,)), _sd((NPAD,)), _sd((NPAD,)), _sd((NW * NSUM,))),
    scratch_types=[
        pltpu.VMEM((NPT2,), _f32),
        pltpu.VMEM((NPT2,), _f32),
        pltpu.VMEM((NPT2,), _f32),
        pltpu.VMEM((NPT2,), _f32),
        pltpu.VMEM((NSUM,), _f32),
        pltpu.VMEM((16,), _f32),
    ])


def _upd_body(t, pacc_h, deg_in, sus_h, rec_h, inf_h, thr_h,
              beta_h, gamma_h,
              inf2_h, sus2_h, rec2_h, *rest):
    if t == 0:
        deg_h, sums_h = rest[0], rest[1]
        rest = rest[2:]
    else:
        sums_h = rest[0]
        rest = rest[1:]
    (acc0, acc1, deg_l, sus_l, rec_l, inf_l, thra, thrb,
     sums_v, beta_v, gamma_v) = rest

    cid = lax.axis_index("c")
    sid = lax.axis_index("s")
    wid = cid * NT + sid
    nb = wid * NPT2

    pltpu.sync_copy(pacc_h.at[pl.ds(nb, NPT2)], acc0)
    pltpu.sync_copy(pacc_h.at[pl.ds(NPAD + nb, NPT2)], acc1)
    if t == 0:
        pltpu.sync_copy(deg_in.at[pl.ds(nb, NPT2)], deg_l)
        pltpu.sync_copy(deg_in.at[pl.ds(NPAD + nb, NPT2)], thra)
    else:
        pltpu.sync_copy(deg_in.at[pl.ds(nb, NPT2)], deg_l)
    pltpu.sync_copy(sus_h.at[pl.ds(nb, NPT2)], sus_l)
    pltpu.sync_copy(rec_h.at[pl.ds(nb, NPT2)], rec_l)
    pltpu.sync_copy(inf_h.at[pl.ds(nb, NPT2)], inf_l)
    pltpu.sync_copy(beta_h, beta_v)
    pltpu.sync_copy(gamma_h, gamma_v)

    if t == 0:
        def db(i, car):
            ix = pl.ds(i * 16, 16)
            deg_l[ix] = deg_l[ix] + thra[ix]
            return car
        lax.fori_loop(0, NVEC2, db, 0)
        pltpu.sync_copy(deg_l, deg_h.at[pl.ds(nb, NPT2)])

    pltpu.sync_copy(thr_h.at[pl.ds((2 * t + 1) * NPAD + nb, NPT2)], thra)
    pltpu.sync_copy(thr_h.at[pl.ds((2 * t + 2) * NPAD + nb, NPT2)], thrb)

    bv = beta_v[...]
    gv = gamma_v[...]

    def b(i, car):
        ssus, sinf, srec = car
        ix = pl.ds(i * 16, 16)
        a = acc0[ix] + acc1[ix]
        d = deg_l[ix]
        s = sus_l[ix]
        f = inf_l[ix]
        r = rec_l[ix]
        p = 1.0 - jnp.exp(-(bv * (s * a)) / d)
        p = jnp.minimum(jnp.maximum(p, 1e-10), 1.0)
        ni = jnp.where(p >= thra[ix], 1.0, 0.0)
        pr = jnp.minimum(jnp.maximum(gv * f, 1e-10), 1.0)
        nr = jnp.where(pr >= thrb[ix], 1.0, 0.0)
        f2 = f + ni - nr
        s2 = s - ni
        r2 = r + nr
        inf_l[ix] = f2
        sus_l[ix] = s2
        rec_l[ix] = r2
        return (ssus + s2, sinf + f2, srec + r2)

    z16 = jnp.zeros((16,), _f32)
    ssus, sinf, srec = lax.fori_loop(0, NVEC2, b, (z16, z16, z16))
    sums_v[pl.ds(0, 16)] = ssus
    sums_v[pl.ds(16, 16)] = sinf
    sums_v[pl.ds(32, 16)] = srec
    pltpu.sync_copy(inf_l, inf2_h.at[pl.ds(nb, NPT2)])
    pltpu.sync_copy(sus_l, sus2_h.at[pl.ds(nb, NPT2)])
    pltpu.sync_copy(rec_l, rec2_h.at[pl.ds(nb, NPT2)])
    pltpu.sync_copy(sums_v, sums_h.at[pl.ds(wid * NSUM, NSUM)])


def _upd_scratch():
    return [
        pltpu.VMEM((NPT2,), _f32),
        pltpu.VMEM((NPT2,), _f32),
        pltpu.VMEM((NPT2,), _f32),
        pltpu.VMEM((NPT2,), _f32),
        pltpu.VMEM((NPT2,), _f32),
        pltpu.VMEM((NPT2,), _f32),
        pltpu.VMEM((NPT2,), _f32),
        pltpu.VMEM((NPT2,), _f32),
        pltpu.VMEM((NSUM,), _f32),
        pltpu.VMEM((16,), _f32),
        pltpu.VMEM((16,), _f32),
    ]


_upd0 = pl.kernel(
    lambda *a: _upd_body(0, *a), mesh=_mesh,
    out_type=(_sd((NPAD,)), _sd((NPAD,)), _sd((NPAD,)), _sd((NPAD,)),
              _sd((NW * NSUM,))),
    scratch_types=_upd_scratch())

_upds = [
    pl.kernel(
        lambda *a, _t=t: _upd_body(_t, *a), mesh=_mesh,
        out_type=(_sd((NPAD,)), _sd((NPAD,)), _sd((NPAD,)),
                  _sd((NW * NSUM,))),
        scratch_types=_upd_scratch())
    for t in range(1, N_STEPS)
]


@jax.jit
def _run(src_pad, dst_pad, thr, beta16, gamma16, frac16):
    inf, sus, rec, s0 = _init(thr, frac16)
    day_sums = [s0]
    pacc, pdeg = _edge0(src_pad, dst_pad, inf)
    inf, sus, rec, deg, s1 = _upd0(pacc, pdeg, sus, rec, inf, thr,
                                   beta16, gamma16)
    day_sums.append(s1)
    for t in range(1, N_STEPS):
        pacc = _edge(src_pad, dst_pad, inf)
        inf, sus, rec, st = _upds[t - 1](pacc, deg, sus, rec, inf, thr,
                                         beta16, gamma16)
        day_sums.append(st)
    return jnp.stack(day_sums)


def kernel(params, edge_index):
    beta = params[0]
    gamma = params[1]
    frac0 = params[2]

    noise_key = jax.random.key(1234)
    keys = jax.vmap(lambda i: jax.random.fold_in(noise_key, i))(
        jnp.arange(2 * N_STEPS + 1))
    g = jax.vmap(lambda k: jax.random.gumbel(k, (N_AGENTS, 2),
                                             dtype=jnp.float32))(keys)
    thr = jax.nn.sigmoid(g[:, :, 1] - g[:, :, 0])
    thr = jnp.pad(thr, ((0, 0), (0, NPAD - N_AGENTS)), constant_values=2.0)
    thr = thr.reshape(-1)

    src = edge_index[0]
    dst = edge_index[1]
    npad_e = EPAD - N_EDGES
    src_pad = jnp.concatenate([src, jnp.zeros((npad_e,), jnp.int32)])
    dst_pad = jnp.concatenate(
        [dst, jnp.full((npad_e,), N_AGENTS, jnp.int32)])

    beta16 = jnp.full((16,), beta, jnp.float32)
    gamma16 = jnp.full((16,), gamma, jnp.float32)
    frac16 = jnp.full((16,), frac0, jnp.float32)

    day_sums = _run(src_pad, dst_pad, thr, beta16, gamma16, frac16)
    totals = jnp.sum(day_sums.reshape(N_STEPS + 1, NW, 3, 16),
                     axis=(1, 3))
    sus_days = totals[:, 0] / N_AGENTS
    inf_days = totals[:, 1] / N_AGENTS
    rec_days = totals[:, 2] / N_AGENTS
    return (sus_days, inf_days, rec_days)


# speedup vs baseline: 1.9643x; 1.9643x over previous
import jax
import jax.numpy as jnp
from jax import lax
from jax.experimental import pallas as pl
from jax.experimental.pallas import tpu as pltpu
from jax.experimental.pallas import tpu_sc as plsc

N_AGENTS = 100000
N_EDGES = 3200000
N_STEPS = 10

NC = 2
NT = 16
NW = NC * NT
NPT = 6272
NPAD = NT * NPT
NPT2 = NPAD // NW
NVEC2 = NPT2 // 16
EPW = 100352
EPAD = NW * EPW
CH = 3584
NCHUNK = EPW // CH
NOUT = NCHUNK // 4
NSUM = 3 * 16

_mesh = plsc.VectorSubcoreMesh(core_axis_name="c", subcore_axis_name="s")
_f32 = jnp.float32


def _sd(shape):
    return jax.ShapeDtypeStruct(shape, _f32)


def _edge_body(with_deg, src_h, dst_h, inf_h, pacc_h, *rest):
    if with_deg:
        pdeg_h, *rest = rest
    (sidx0, didx0, sidx1, didx1, sidx2, didx2, sidx3, didx3,
     vals0, vals1, ones_v, zero_l, inf_s, acc_s, deg_s,
     sem0, sem1, sem2, sem3, ssc0, ssc1) = rest

    cid = lax.axis_index("c")
    sid = lax.axis_index("s")
    wid = cid * NT + sid
    nbase = sid * NPT
    ebase = wid * EPW

    def fill(ref, n, val):
        def b(i, car):
            ref[pl.ds(i * 16, 16)] = jnp.full((16,), val, _f32)
            return car
        lax.fori_loop(0, n // 16, b, 0)

    if with_deg:
        fill(ones_v, CH, 1.0)
    fill(zero_l, NPT, 0.0)

    pltpu.sync_copy(zero_l, acc_s.at[pl.ds(nbase, NPT)])
    if with_deg:
        pltpu.sync_copy(zero_l, deg_s.at[pl.ds(nbase, NPT)])
    pltpu.sync_copy(inf_h.at[pl.ds(nbase, NPT)], zero_l)
    pltpu.sync_copy(zero_l, inf_s.at[pl.ds(nbase, NPT)])
    plsc.subcore_barrier()

    ibufs = [(sidx0, didx0, sem0), (sidx1, didx1, sem1),
             (sidx2, didx2, sem2), (sidx3, didx3, sem3)]
    vbufs = [(vals0, ssc0), (vals1, ssc1)]

    def start_idx(c, b):
        sb, db, sem = ibufs[b]
        base = ebase + c * CH
        pltpu.make_async_copy(src_h.at[pl.ds(base, CH)], sb, sem).start()
        pltpu.make_async_copy(dst_h.at[pl.ds(base, CH)], db, sem).start()

    def wait_idx(c, b):
        sb, db, sem = ibufs[b]
        base = ebase + c * CH
        pltpu.make_async_copy(src_h.at[pl.ds(base, CH)], sb, sem).wait()
        pltpu.make_async_copy(dst_h.at[pl.ds(base, CH)], db, sem).wait()

    def wait_sc(v):
        vb, sem = vbufs[v]
        pltpu.make_async_copy(vb, acc_s.at[ibufs[0][1]], sem).wait()

    start_idx(0, 0)
    start_idx(1, 1)

    def edge_loop(co, car):
        for u in range(4):
            e = 4 * co + u
            vb, vsem = vbufs[u % 2]

            @pl.when(e >= 2)
            def _():
                wait_sc(u % 2)

            wait_idx(e, u)
            pltpu.sync_copy(inf_s.at[ibufs[u][0]], vb)
            pltpu.make_async_copy(
                vb, acc_s.at[ibufs[u][1]], vsem).start(add=True)
            if with_deg:
                pltpu.sync_copy(ones_v, deg_s.at[ibufs[u][1]], add=True)

            @pl.when(e + 2 < NCHUNK)
            def _():
                start_idx(e + 2, (u + 2) % 4)
        return car

    lax.fori_loop(0, NOUT, edge_loop, 0)
    wait_sc(0)
    wait_sc(1)
    plsc.subcore_barrier()

    pltpu.sync_copy(acc_s.at[pl.ds(nbase, NPT)],
                    pacc_h.at[pl.ds(cid * NPAD + nbase, NPT)])
    if with_deg:
        pltpu.sync_copy(deg_s.at[pl.ds(nbase, NPT)],
                        pdeg_h.at[pl.ds(cid * NPAD + nbase, NPT)])


_edge_scratch = [
    pltpu.VMEM((CH,), jnp.int32),
    pltpu.VMEM((CH,), jnp.int32),
    pltpu.VMEM((CH,), jnp.int32),
    pltpu.VMEM((CH,), jnp.int32),
    pltpu.VMEM((CH,), jnp.int32),
    pltpu.VMEM((CH,), jnp.int32),
    pltpu.VMEM((CH,), jnp.int32),
    pltpu.VMEM((CH,), jnp.int32),
    pltpu.VMEM((CH,), _f32),
    pltpu.VMEM((CH,), _f32),
    pltpu.VMEM((CH,), _f32),
    pltpu.VMEM((NPT,), _f32),
    pltpu.VMEM_SHARED((NPAD,), _f32),
    pltpu.VMEM_SHARED((NPAD,), _f32),
    pltpu.VMEM_SHARED((NPAD,), _f32),
    pltpu.SemaphoreType.DMA,
    pltpu.SemaphoreType.DMA,
    pltpu.SemaphoreType.DMA,
    pltpu.SemaphoreType.DMA,
    pltpu.SemaphoreType.DMA,
    pltpu.SemaphoreType.DMA,
]

_edge0 = pl.kernel(
    lambda *a: _edge_body(True, *a), mesh=_mesh,
    out_type=(_sd((NC * NPAD,)), _sd((NC * NPAD,))),
    scratch_types=_edge_scratch)

_edge = pl.kernel(
    lambda *a: _edge_body(False, *a), mesh=_mesh,
    out_type=_sd((NC * NPAD,)),
    scratch_types=_edge_scratch)


def _init_body(thr_h, frac_h, inf_h, sus_h, rec_h, sums_h,
               thra, inf_l, sus_l, rec_l, sums_v, frac_v):
    cid = lax.axis_index("c")
    sid = lax.axis_index("s")
    wid = cid * NT + sid
    nb = wid * NPT2

    pltpu.sync_copy(thr_h.at[pl.ds(nb, NPT2)], thra)
    pltpu.sync_copy(frac_h, frac_v)
    fv = frac_v[...]

    def b(i, car):
        ssus, sinf = car
        t0 = thra[pl.ds(i * 16, 16)]
        inf = jnp.where(fv >= t0, 1.0, 0.0)
        sus = jnp.where(t0 > 1.5, 0.0, 1.0 - inf)
        inf_l[pl.ds(i * 16, 16)] = inf
        sus_l[pl.ds(i * 16, 16)] = sus
        rec_l[pl.ds(i * 16, 16)] = jnp.zeros((16,), _f32)
        return (ssus + sus, sinf + inf)

    z16 = jnp.zeros((16,), _f32)
    ssus, sinf = lax.fori_loop(0, NVEC2, b, (z16, z16))
    sums_v[pl.ds(0, 16)] = ssus
    sums_v[pl.ds(16, 16)] = sinf
    sums_v[pl.ds(32, 16)] = z16
    pltpu.sync_copy(inf_l, inf_h.at[pl.ds(nb, NPT2)])
    pltpu.sync_copy(sus_l, sus_h.at[pl.ds(nb, NPT2)])
    pltpu.sync_copy(rec_l, rec_h.at[pl.ds(nb, NPT2)])
    pltpu.sync_copy(sums_v, sums_h.at[pl.ds(wid * NSUM, NSUM)])


_init = pl.kernel(
    _init_body, mesh=_mesh,
    out_type=(_sd((NPAD,)), _sd((NPAD,)), _sd((NPAD,)), _sd((NW * NSUM,))),
    scratch_types=[
        pltpu.VMEM((NPT2,), _f32),
        pltpu.VMEM((NPT2,), _f32),
        pltpu.VMEM((NPT2,), _f32),
        pltpu.VMEM((NPT2,), _f32),
        pltpu.VMEM((NSUM,), _f32),
        pltpu.VMEM((16,), _f32),
    ])


def _upd_body(t, pacc_h, deg_in, sus_h, rec_h, inf_h, thr_h,
              beta_h, gamma_h,
              inf2_h, sus2_h, rec2_h, *rest):
    if t == 0:
        deg_h, sums_h = rest[0], rest[1]
        rest = rest[2:]
    else:
        sums_h = rest[0]
        rest = rest[1:]
    (acc0, acc1, deg_l, sus_l, rec_l, inf_l, thra, thrb,
     sums_v, beta_v, gamma_v) = rest

    cid = lax.axis_index("c")
    sid = lax.axis_index("s")
    wid = cid * NT + sid
    nb = wid * NPT2

    pltpu.sync_copy(pacc_h.at[pl.ds(nb, NPT2)], acc0)
    pltpu.sync_copy(pacc_h.at[pl.ds(NPAD + nb, NPT2)], acc1)
    if t == 0:
        pltpu.sync_copy(deg_in.at[pl.ds(nb, NPT2)], deg_l)
        pltpu.sync_copy(deg_in.at[pl.ds(NPAD + nb, NPT2)], thra)
    else:
        pltpu.sync_copy(deg_in.at[pl.ds(nb, NPT2)], deg_l)
    pltpu.sync_copy(sus_h.at[pl.ds(nb, NPT2)], sus_l)
    pltpu.sync_copy(rec_h.at[pl.ds(nb, NPT2)], rec_l)
    pltpu.sync_copy(inf_h.at[pl.ds(nb, NPT2)], inf_l)
    pltpu.sync_copy(beta_h, beta_v)
    pltpu.sync_copy(gamma_h, gamma_v)

    if t == 0:
        def db(i, car):
            ix = pl.ds(i * 16, 16)
            deg_l[ix] = deg_l[ix] + thra[ix]
            return car
        lax.fori_loop(0, NVEC2, db, 0)
        pltpu.sync_copy(deg_l, deg_h.at[pl.ds(nb, NPT2)])

    pltpu.sync_copy(thr_h.at[pl.ds((2 * t + 1) * NPAD + nb, NPT2)], thra)
    pltpu.sync_copy(thr_h.at[pl.ds((2 * t + 2) * NPAD + nb, NPT2)], thrb)

    bv = beta_v[...]
    gv = gamma_v[...]

    def b(i, car):
        ssus, sinf, srec = car
        ix = pl.ds(i * 16, 16)
        a = acc0[ix] + acc1[ix]
        d = deg_l[ix]
        s = sus_l[ix]
        f = inf_l[ix]
        r = rec_l[ix]
        p = 1.0 - jnp.exp(-(bv * (s * a)) / d)
        p = jnp.minimum(jnp.maximum(p, 1e-10), 1.0)
        ni = jnp.where(p >= thra[ix], 1.0, 0.0)
        pr = jnp.minimum(jnp.maximum(gv * f, 1e-10), 1.0)
        nr = jnp.where(pr >= thrb[ix], 1.0, 0.0)
        f2 = f + ni - nr
        s2 = s - ni
        r2 = r + nr
        inf_l[ix] = f2
        sus_l[ix] = s2
        rec_l[ix] = r2
        return (ssus + s2, sinf + f2, srec + r2)

    z16 = jnp.zeros((16,), _f32)
    ssus, sinf, srec = lax.fori_loop(0, NVEC2, b, (z16, z16, z16))
    sums_v[pl.ds(0, 16)] = ssus
    sums_v[pl.ds(16, 16)] = sinf
    sums_v[pl.ds(32, 16)] = srec
    pltpu.sync_copy(inf_l, inf2_h.at[pl.ds(nb, NPT2)])
    pltpu.sync_copy(sus_l, sus2_h.at[pl.ds(nb, NPT2)])
    pltpu.sync_copy(rec_l, rec2_h.at[pl.ds(nb, NPT2)])
    pltpu.sync_copy(sums_v, sums_h.at[pl.ds(wid * NSUM, NSUM)])


def _upd_scratch():
    return [
        pltpu.VMEM((NPT2,), _f32),
        pltpu.VMEM((NPT2,), _f32),
        pltpu.VMEM((NPT2,), _f32),
        pltpu.VMEM((NPT2,), _f32),
        pltpu.VMEM((NPT2,), _f32),
        pltpu.VMEM((NPT2,), _f32),
        pltpu.VMEM((NPT2,), _f32),
        pltpu.VMEM((NPT2,), _f32),
        pltpu.VMEM((NSUM,), _f32),
        pltpu.VMEM((16,), _f32),
        pltpu.VMEM((16,), _f32),
    ]


_upd0 = pl.kernel(
    lambda *a: _upd_body(0, *a), mesh=_mesh,
    out_type=(_sd((NPAD,)), _sd((NPAD,)), _sd((NPAD,)), _sd((NPAD,)),
              _sd((NW * NSUM,))),
    scratch_types=_upd_scratch())

_upds = [
    pl.kernel(
        lambda *a, _t=t: _upd_body(_t, *a), mesh=_mesh,
        out_type=(_sd((NPAD,)), _sd((NPAD,)), _sd((NPAD,)),
                  _sd((NW * NSUM,))),
        scratch_types=_upd_scratch())
    for t in range(1, N_STEPS)
]


@jax.jit
def _run(src_pad, dst_pad, thr, beta16, gamma16, frac16):
    inf, sus, rec, s0 = _init(thr, frac16)
    day_sums = [s0]
    pacc, pdeg = _edge0(src_pad, dst_pad, inf)
    inf, sus, rec, deg, s1 = _upd0(pacc, pdeg, sus, rec, inf, thr,
                                   beta16, gamma16)
    day_sums.append(s1)
    for t in range(1, N_STEPS):
        pacc = _edge(src_pad, dst_pad, inf)
        inf, sus, rec, st = _upds[t - 1](pacc, deg, sus, rec, inf, thr,
                                         beta16, gamma16)
        day_sums.append(st)
    return jnp.stack(day_sums)


def kernel(params, edge_index):
    beta = params[0]
    gamma = params[1]
    frac0 = params[2]

    noise_key = jax.random.key(1234)
    keys = jax.vmap(lambda i: jax.random.fold_in(noise_key, i))(
        jnp.arange(2 * N_STEPS + 1))
    g = jax.vmap(lambda k: jax.random.gumbel(k, (N_AGENTS, 2),
                                             dtype=jnp.float32))(keys)
    thr = jax.nn.sigmoid(g[:, :, 1] - g[:, :, 0])
    thr = jnp.pad(thr, ((0, 0), (0, NPAD - N_AGENTS)), constant_values=2.0)
    thr = thr.reshape(-1)

    src = edge_index[0]
    dst = edge_index[1]
    npad_e = EPAD - N_EDGES
    src_pad = jnp.concatenate([src, jnp.zeros((npad_e,), jnp.int32)])
    dst_pad = jnp.concatenate(
        [dst, jnp.full((npad_e,), N_AGENTS, jnp.int32)])

    beta16 = jnp.full((16,), beta, jnp.float32)
    gamma16 = jnp.full((16,), gamma, jnp.float32)
    frac16 = jnp.full((16,), frac0, jnp.float32)

    day_sums = _run(src_pad, dst_pad, thr, beta16, gamma16, frac16)
    totals = jnp.sum(day_sums.reshape(N_STEPS + 1, NW, 3, 16),
                     axis=(1, 3))
    sus_days = totals[:, 0] / N_AGENTS
    inf_days = totals[:, 1] / N_AGENTS
    rec_days = totals[:, 2] / N_AGENTS
    return (sus_days, inf_days, rec_days)
